# baseline (speedup 1.0000x reference)
import functools

import jax
import jax.numpy as jnp
from jax import lax
from jax.experimental import pallas as pl
from jax.experimental.pallas import tpu as pltpu
from jax.experimental.pallas import tpu_sc as plsc

N = 10000
E = 160000
D_IN = 256
D_HID = 512
NC, NS, LANES = 2, 16, 16
NW = NC * NS

B_PAD = 10240
B_W = B_PAD // NW
GCH = 64
NCH = B_W // GCH

EP = 163840
ECH = 64
EW = EP // NS
NWIN = EW // ECH
NBUF = 4
NACC = 10240
RST = NACC // NS

BLK = 400
GRID = N // BLK


def _sc_mesh():
    return plsc.VectorSubcoreMesh(core_axis_name="c", subcore_axis_name="s",
                                  num_cores=NC, num_subcores=NS)


@functools.lru_cache(maxsize=None)
def _get_gather():
    @functools.partial(
        pl.kernel,
        out_type=(
            jax.ShapeDtypeStruct((B_PAD, D_IN), jnp.float32),
            jax.ShapeDtypeStruct((B_PAD, 128), jnp.float32),
        ),
        mesh=_sc_mesh(),
        scratch_types=[
            pltpu.VMEM((B_W,), jnp.int32),
            pltpu.VMEM((GCH, D_IN), jnp.float32),
            pltpu.VMEM((GCH, 128), jnp.float32),
            pltpu.SemaphoreType.DMA,
            pltpu.SemaphoreType.DMA,
        ],
    )
    def _gather_k(feats, labels, idx1d, x_out, lg_out, idxv, fbuf, lbuf, s1,
                  s2):
        c = lax.axis_index("c")
        sid = lax.axis_index("s")
        wid = sid * NC + c
        base = wid * B_W
        pltpu.sync_copy(idx1d.at[pl.ds(base, B_W)], idxv)
        for ch in range(NCH):
            d1 = pltpu.async_copy(feats.at[idxv.at[pl.ds(ch * GCH, GCH)]],
                                  fbuf, s1)
            d2 = pltpu.async_copy(labels.at[idxv.at[pl.ds(ch * GCH, GCH)]],
                                  lbuf, s2)
            d1.wait()
            d2.wait()
            pltpu.sync_copy(fbuf, x_out.at[pl.ds(base + ch * GCH, GCH)])
            pltpu.sync_copy(lbuf, lg_out.at[pl.ds(base + ch * GCH, GCH)])

    return _gather_k


@functools.lru_cache(maxsize=None)
def _make_spmm(S, T):
    NP = S // NC

    @functools.partial(
        pl.kernel,
        out_type=jax.ShapeDtypeStruct((S, NACC, 128), jnp.float32),
        mesh=_sc_mesh(),
        scratch_types=(
            [pltpu.VMEM_SHARED((NACC, 128), jnp.float32),
             pltpu.VMEM((EW,), jnp.int32)]
            + [pltpu.VMEM((ECH, 128), jnp.float32)] * NBUF
            + [pltpu.VMEM((1, ECH), jnp.int32)] * NBUF
            + [pltpu.VMEM((ECH,), jnp.float32)] * NBUF
            + [pltpu.SemaphoreType.DMA] * (4 * NBUF)
        ),
    )
    def spmm_k(tbl, src1, dst3, val1, out, acc, idxv, *bufs):
        gb = bufs[0:NBUF]
        db = bufs[NBUF:2 * NBUF]
        vb = bufs[2 * NBUF:3 * NBUF]
        gs = bufs[3 * NBUF:4 * NBUF]
        ss = bufs[4 * NBUF:5 * NBUF]
        ds = bufs[5 * NBUF:6 * NBUF]
        vs = bufs[6 * NBUF:7 * NBUF]
        c = lax.axis_index("c")
        sid = lax.axis_index("s")
        e0 = sid * EW
        w0 = sid * NWIN

        zero16 = jnp.zeros((LANES,), jnp.float32)
        r0 = sid * RST

        def start_window(k, j):
            pltpu.async_copy(tbl.at[idxv.at[pl.ds(j * ECH, ECH)]], gb[k],
                             gs[k])
            pltpu.async_copy(dst3.at[w0 + j], db[k], ds[k])
            pltpu.async_copy(val1.at[pl.ds(e0 + j * ECH, ECH)], vb[k], vs[k])

        def wait_window(k):
            pltpu.make_async_copy(tbl.at[idxv.at[pl.ds(0, ECH)]], gb[k],
                                  gs[k]).wait()
            pltpu.make_async_copy(dst3.at[w0], db[k], ds[k]).wait()
            pltpu.make_async_copy(val1.at[pl.ds(0, ECH)], vb[k], vs[k]).wait()

        def start_scatter(k):
            pltpu.async_copy(gb[k], acc.at[db[k].at[0]], ss[k], add=True)

        def wait_scatter(k):
            pltpu.make_async_copy(gb[k], acc.at[db[k].at[0]], ss[k]).wait()

        def scale(k):
            buf = gb[k]
            vbk = vb[k]

            def eb(g, ecarry):
                v16 = vbk[pl.ds(g * LANES, LANES)]
                for i in range(LANES):
                    e = g * LANES + i
                    sv = v16[i]
                    for kk in range(8):
                        buf[e, pl.ds(kk * LANES, LANES)] = (
                            buf[e, pl.ds(kk * LANES, LANES)] * sv)
                return ecarry

            lax.fori_loop(0, ECH // LANES, eb, 0)

        for p in range(NP):
            strip = c * NP + p

            def zb(t, carry):
                for k in range(8):
                    gb[0][t, pl.ds(k * LANES, LANES)] = zero16
                return carry

            lax.fori_loop(0, ECH, zb, 0)
            for q in range(RST // ECH):
                pltpu.sync_copy(gb[0], acc.at[pl.ds(r0 + q * ECH, ECH)])
            plsc.subcore_barrier()

            pltpu.sync_copy(src1.at[pl.ds(e0, EW)], idxv)

            def ib(j, carry):
                for k in range(4):
                    t = j * 4 + k
                    idxv[pl.ds(t * LANES, LANES)] = (
                        idxv[pl.ds(t * LANES, LANES)] * S + strip)
                return carry

            lax.fori_loop(0, EW // (4 * LANES), ib, 0)

            start_window(0, 0)
            start_window(1, 1)

            def cb(t, carry):
                for k in range(NBUF):
                    j = t * NBUF + k
                    m = (k + 2) % NBUF

                    @pl.when(j >= 2)
                    def _():
                        wait_scatter(m)

                    @pl.when(j + 2 < NWIN)
                    def _():
                        start_window(m, j + 2)

                    wait_window(k)
                    scale(k)
                    start_scatter(k)
                return carry

            lax.fori_loop(0, NWIN // NBUF, cb, 0)
            wait_scatter((NWIN - 2) % NBUF)
            wait_scatter((NWIN - 1) % NBUF)
            plsc.subcore_barrier()
            pltpu.sync_copy(acc.at[pl.ds(r0, RST)],
                            out.at[strip, pl.ds(r0, RST)])
            if p < NP - 1:
                plsc.subcore_barrier()

    return spmm_k


def _ln(f, s, o):
    f = jnp.maximum(f, 0.0)
    mean = jnp.mean(f, axis=1, keepdims=True)
    var = jnp.mean((f - mean) ** 2, axis=1, keepdims=True) + 1e-9
    return (f - mean) * s * lax.rsqrt(var) + o


_LG_W = 128


def _tcb_body(x_ref, lg_ref, x1_ref, w00_ref, b00_ref, s00_ref, o00_ref,
              iota_ref, w01_ref, b01_ref, s01_ref, o01_ref,
              w10_ref, b10_ref, s10_ref, o10_ref, w11_ref,
              h0p_ref, y_ref, am_ref):
    x = x_ref[...]
    f0 = (jnp.dot(x, w00_ref[...], preferred_element_type=jnp.float32)
          + b00_ref[...])
    h0 = _ln(f0, s00_ref[...], o00_ref[...])
    am_ref[...] = jnp.sum(lg_ref[...] * iota_ref[...], axis=1, keepdims=True)
    w01 = w01_ref[...]
    f1 = (jnp.dot(x1_ref[0], w01[:128], preferred_element_type=jnp.float32)
          + jnp.dot(x1_ref[1], w01[128:], preferred_element_type=jnp.float32)
          + b01_ref[...])
    h1 = _ln(f1, s01_ref[...], o01_ref[...])
    w10 = w10_ref[...]
    f2 = (jnp.dot(h0, w10[:D_HID], preferred_element_type=jnp.float32)
          + jnp.dot(h1, w10[D_HID:], preferred_element_type=jnp.float32)
          + b10_ref[...])
    h0p_ref[...] = _ln(f2, s10_ref[...], o10_ref[...])
    w11 = w11_ref[...]
    y_ref[...] = (jnp.dot(h0, w11[:D_HID], preferred_element_type=jnp.float32)
                  + jnp.dot(h1, w11[D_HID:],
                            preferred_element_type=jnp.float32))


def _tcc_body(z_ref, h0p_ref, b11_ref, s11_ref, o11_ref, w2_ref, b2_ref,
              pred_ref):
    b11 = b11_ref[...]
    s11 = s11_ref[...]
    o11 = o11_ref[...]
    ts = [jnp.maximum(z_ref[s] + b11[:, s * 128:(s + 1) * 128], 0.0)
          for s in range(4)]
    sum1 = sum(jnp.sum(t, axis=1, keepdims=True) for t in ts)
    sum2 = sum(jnp.sum(t * t, axis=1, keepdims=True) for t in ts)
    mean = sum1 / D_HID
    var = sum2 / D_HID - mean * mean + 1e-9
    rstd = lax.rsqrt(var)
    h1ps = [(ts[s] - mean) * s11[:, s * 128:(s + 1) * 128] * rstd
            + o11[:, s * 128:(s + 1) * 128] for s in range(4)]
    h0p = h0p_ref[...]
    nsq = jnp.sum(h0p * h0p, axis=1, keepdims=True)
    for t in h1ps:
        nsq = nsq + jnp.sum(t * t, axis=1, keepdims=True)
    inv = 1.0 / jnp.maximum(jnp.sqrt(nsq), 1e-12)
    w2 = w2_ref[...]
    acc = jnp.dot(h0p, w2[:D_HID], preferred_element_type=jnp.float32)
    for s in range(4):
        acc = acc + jnp.dot(h1ps[s], w2[D_HID + s * 128:D_HID + (s + 1) * 128],
                            preferred_element_type=jnp.float32)
    pred_ref[...] = acc * inv + b2_ref[...]


def _row_spec(shape):
    nd = len(shape)
    return pl.BlockSpec(shape, lambda i: (i,) + (0,) * (nd - 1))


def _const_spec(shape):
    nd = len(shape)
    return pl.BlockSpec(shape, lambda i: (0,) * nd)


def _tcb(x, lg, x1s, w00, b00, s00, o00, iota_row, w01, b01, s01, o01,
         w10, b10, s10, o10, w11):
    return pl.pallas_call(
        _tcb_body,
        grid=(GRID,),
        in_specs=[
            _row_spec((BLK, D_IN)),
            _row_spec((BLK, _LG_W)),
            pl.BlockSpec((2, BLK, 128), lambda i: (0, i, 0)),
            _const_spec((D_IN, D_HID)),
            _const_spec((1, D_HID)),
            _const_spec((1, D_HID)),
            _const_spec((1, D_HID)),
            _const_spec((1, _LG_W)),
            _const_spec((D_IN, D_HID)),
            _const_spec((1, D_HID)),
            _const_spec((1, D_HID)),
            _const_spec((1, D_HID)),
            _const_spec((2 * D_HID, D_HID)),
            _const_spec((1, D_HID)),
            _const_spec((1, D_HID)),
            _const_spec((1, D_HID)),
            _const_spec((2 * D_HID, D_HID)),
        ],
        out_specs=[_row_spec((BLK, D_HID)), _row_spec((BLK, D_HID)),
                   _row_spec((BLK, 1))],
        out_shape=[
            jax.ShapeDtypeStruct((N, D_HID), jnp.float32),
            jax.ShapeDtypeStruct((N, D_HID), jnp.float32),
            jax.ShapeDtypeStruct((N, 1), jnp.float32),
        ],
    )(x, lg, x1s, w00, b00, s00, o00, iota_row, w01, b01, s01, o01,
      w10, b10, s10, o10, w11)


def _tcc(zs, h0p, b11, s11, o11, w2, b2):
    return pl.pallas_call(
        _tcc_body,
        grid=(GRID,),
        in_specs=[
            pl.BlockSpec((4, BLK, 128), lambda i: (0, i, 0)),
            _row_spec((BLK, D_HID)),
            _const_spec((1, D_HID)),
            _const_spec((1, D_HID)),
            _const_spec((1, D_HID)),
            _const_spec((2 * D_HID, 64)),
            _const_spec((1, 64)),
        ],
        out_specs=_row_spec((BLK, 64)),
        out_shape=jax.ShapeDtypeStruct((N, 64), jnp.float32),
    )(zs, h0p, b11, s11, o11, w2, b2)


def kernel(node_ids, edge_index, adj_vals, feats, labels,
           W0_0, b0_0, s0_0, o0_0, W0_1, b0_1, s0_1, o0_1,
           W1_0, b1_0, s1_0, o1_0, W1_1, b1_1, s1_1, o1_1,
           W2, b2):
    f32 = jnp.float32
    i32 = jnp.int32
    src = edge_index[1]
    dst = edge_index[0]
    npad = EP - E
    pad_idx = (jnp.arange(npad, dtype=i32) * 37) % N
    src1 = jnp.concatenate([src, pad_idx])
    dst3 = jnp.concatenate([dst, pad_idx]).reshape(EP // ECH, 1, ECH)
    val1 = jnp.concatenate([adj_vals, jnp.zeros((npad,), f32)])
    nid_pad = (jnp.arange(B_PAD - N, dtype=i32) * 41) % N
    nid1 = jnp.concatenate([node_ids, nid_pad])

    labels128 = jnp.pad(labels, ((0, 0), (0, 64)))
    x_pad, lg = _get_gather()(feats, labels128, nid1)

    iota_row = jnp.arange(_LG_W, dtype=f32).reshape(1, _LG_W)
    x1s = _make_spmm(2, B_PAD)(x_pad.reshape(B_PAD * 2, 128), src1, dst3,
                               val1)

    h0p, y, am = _tcb(x_pad, lg, x1s, W0_0, b0_0.reshape(1, -1),
                      s0_0.reshape(1, -1), o0_0.reshape(1, -1), iota_row,
                      W0_1, b0_1.reshape(1, -1), s0_1.reshape(1, -1),
                      o0_1.reshape(1, -1), W1_0, b1_0.reshape(1, -1),
                      s1_0.reshape(1, -1), o1_0.reshape(1, -1), W1_1)

    zs = _make_spmm(4, N)(y.reshape(N * 4, 128), src1, dst3, val1)

    pred = _tcc(zs, h0p, b1_1.reshape(1, -1), s1_1.reshape(1, -1),
                o1_1.reshape(1, -1), W2, b2.reshape(1, -1))
    lab = am[:, 0].astype(i32)
    return (pred, lab)


# speedup vs baseline: 1.0674x; 1.0674x over previous
import functools

import jax
import jax.numpy as jnp
from jax import lax
from jax.experimental import pallas as pl
from jax.experimental.pallas import tpu as pltpu
from jax.experimental.pallas import tpu_sc as plsc

N = 10000
E = 160000
D_IN = 256
D_HID = 512
NC, NS, LANES = 2, 16, 16
NW = NC * NS

B_PAD = 10240
B_W = B_PAD // NW
GCH = 64
NCH = B_W // GCH

EP = 163840
ECH = 64
EW = EP // NS
NWIN = EW // ECH
NBUF = 4
NACC = 10240
RST = NACC // NS

BLK = 400
GRID = N // BLK


def _sc_mesh():
    return plsc.VectorSubcoreMesh(core_axis_name="c", subcore_axis_name="s",
                                  num_cores=NC, num_subcores=NS)


@functools.lru_cache(maxsize=None)
def _get_gather():
    @functools.partial(
        pl.kernel,
        out_type=(
            jax.ShapeDtypeStruct((2, B_PAD, 128), jnp.float32),
            jax.ShapeDtypeStruct((B_PAD, 128), jnp.float32),
        ),
        mesh=_sc_mesh(),
        scratch_types=[
            pltpu.VMEM((B_W,), jnp.int32),
            pltpu.VMEM((GCH, D_IN), jnp.float32),
            pltpu.VMEM((GCH, 128), jnp.float32),
            pltpu.SemaphoreType.DMA,
            pltpu.SemaphoreType.DMA,
        ],
    )
    def _gather_k(feats, labels, idx1d, x_out, lg_out, idxv, fbuf, lbuf, s1,
                  s2):
        c = lax.axis_index("c")
        sid = lax.axis_index("s")
        wid = sid * NC + c
        base = wid * B_W
        pltpu.sync_copy(idx1d.at[pl.ds(base, B_W)], idxv)
        for ch in range(NCH):
            d1 = pltpu.async_copy(feats.at[idxv.at[pl.ds(ch * GCH, GCH)]],
                                  fbuf, s1)
            d2 = pltpu.async_copy(labels.at[idxv.at[pl.ds(ch * GCH, GCH)]],
                                  lbuf, s2)
            d1.wait()
            d2.wait()
            for h in range(2):
                pltpu.sync_copy(
                    fbuf.at[:, pl.ds(h * 128, 128)],
                    x_out.at[h, pl.ds(base + ch * GCH, GCH)])
            pltpu.sync_copy(lbuf, lg_out.at[pl.ds(base + ch * GCH, GCH)])

    return _gather_k


@functools.lru_cache(maxsize=None)
def _make_spmm(S, T):
    NP = S // NC

    @functools.partial(
        pl.kernel,
        out_type=jax.ShapeDtypeStruct((S, NACC, 128), jnp.float32),
        mesh=_sc_mesh(),
        scratch_types=(
            [pltpu.VMEM_SHARED((NACC, 128), jnp.float32),
             pltpu.VMEM((EW,), jnp.int32)]
            + [pltpu.VMEM((ECH, 128), jnp.float32)] * NBUF
            + [pltpu.VMEM((1, ECH), jnp.int32)] * NBUF
            + [pltpu.VMEM((ECH,), jnp.float32)] * NBUF
            + [pltpu.SemaphoreType.DMA] * (4 * NBUF)
        ),
    )
    def spmm_k(tbl, src1, dst3, val1, out, acc, idxv, *bufs):

        gb = bufs[0:NBUF]
        db = bufs[NBUF:2 * NBUF]
        vb = bufs[2 * NBUF:3 * NBUF]
        gs = bufs[3 * NBUF:4 * NBUF]
        ss = bufs[4 * NBUF:5 * NBUF]
        ds = bufs[5 * NBUF:6 * NBUF]
        vs = bufs[6 * NBUF:7 * NBUF]
        c = lax.axis_index("c")
        sid = lax.axis_index("s")
        e0 = sid * EW
        w0 = sid * NWIN

        zero16 = jnp.zeros((LANES,), jnp.float32)
        r0 = sid * RST

        def start_window(k, j):
            pltpu.async_copy(tbl.at[idxv.at[pl.ds(j * ECH, ECH)]], gb[k],
                             gs[k])
            pltpu.async_copy(dst3.at[w0 + j], db[k], ds[k])
            pltpu.async_copy(val1.at[pl.ds(e0 + j * ECH, ECH)], vb[k], vs[k])

        def wait_window(k):
            pltpu.make_async_copy(tbl.at[idxv.at[pl.ds(0, ECH)]], gb[k],
                                  gs[k]).wait()
            pltpu.make_async_copy(dst3.at[w0], db[k], ds[k]).wait()
            pltpu.make_async_copy(val1.at[pl.ds(0, ECH)], vb[k], vs[k]).wait()

        def start_scatter(k):
            pltpu.async_copy(gb[k], acc.at[db[k].at[0]], ss[k], add=True)

        def wait_scatter(k):
            pltpu.make_async_copy(gb[k], acc.at[db[k].at[0]], ss[k]).wait()

        def scale(k):
            buf = gb[k]
            vbk = vb[k]

            def eb(g, ecarry):
                v16 = vbk[pl.ds(g * LANES, LANES)]
                for i in range(LANES):
                    e = g * LANES + i
                    sv = v16[i]
                    for kk in range(8):
                        buf[e, pl.ds(kk * LANES, LANES)] = (
                            buf[e, pl.ds(kk * LANES, LANES)] * sv)
                return ecarry

            lax.fori_loop(0, ECH // LANES, eb, 0)

        for p in range(NP):
            strip = c * NP + p

            def zb(t, carry):
                for k in range(8):
                    gb[0][t, pl.ds(k * LANES, LANES)] = zero16
                return carry

            lax.fori_loop(0, ECH, zb, 0)
            for q in range(RST // ECH):
                pltpu.sync_copy(gb[0], acc.at[pl.ds(r0 + q * ECH, ECH)])
            plsc.subcore_barrier()

            pltpu.sync_copy(src1.at[pl.ds(e0, EW)], idxv)

            def ib(j, carry):
                for k in range(4):
                    t = j * 4 + k
                    idxv[pl.ds(t * LANES, LANES)] = (
                        idxv[pl.ds(t * LANES, LANES)] + strip * T)
                return carry

            lax.fori_loop(0, EW // (4 * LANES), ib, 0)

            start_window(0, 0)
            start_window(1, 1)

            def cb(t, carry):
                for k in range(NBUF):
                    j = t * NBUF + k
                    m = (k + 2) % NBUF

                    @pl.when(j >= 2)
                    def _():
                        wait_scatter(m)

                    @pl.when(j + 2 < NWIN)
                    def _():
                        start_window(m, j + 2)

                    wait_window(k)
                    scale(k)
                    start_scatter(k)
                return carry

            lax.fori_loop(0, NWIN // NBUF, cb, 0)
            wait_scatter((NWIN - 2) % NBUF)
            wait_scatter((NWIN - 1) % NBUF)
            plsc.subcore_barrier()
            pltpu.sync_copy(acc.at[pl.ds(r0, RST)],
                            out.at[strip, pl.ds(r0, RST)])
            if p < NP - 1:
                plsc.subcore_barrier()

    return spmm_k


def _ln(f, s, o):
    f = jnp.maximum(f, 0.0)
    mean = jnp.mean(f, axis=1, keepdims=True)
    var = jnp.mean((f - mean) ** 2, axis=1, keepdims=True) + 1e-9
    return (f - mean) * s * lax.rsqrt(var) + o


def _tca_body(x_ref, lg_ref, w_ref, b_ref, s_ref, o_ref, iota_ref,
              h0_ref, am_ref):
    w = w_ref[...]
    f = (jnp.dot(x_ref[0], w[:128], preferred_element_type=jnp.float32)
         + jnp.dot(x_ref[1], w[128:], preferred_element_type=jnp.float32)
         + b_ref[...])
    h0_ref[...] = _ln(f, s_ref[...], o_ref[...])
    am_ref[...] = jnp.sum(lg_ref[...] * iota_ref[...], axis=1, keepdims=True)


_LG_W = 128


def _tcb_body(x1_ref, h0_ref, w01_ref, b01_ref, s01_ref, o01_ref,
              w10_ref, b10_ref, s10_ref, o10_ref, w11_ref,
              h0p_ref, y_ref):
    w01 = w01_ref[...]
    f1 = (jnp.dot(x1_ref[0], w01[:128], preferred_element_type=jnp.float32)
          + jnp.dot(x1_ref[1], w01[128:], preferred_element_type=jnp.float32)
          + b01_ref[...])
    h1 = _ln(f1, s01_ref[...], o01_ref[...])
    h0 = h0_ref[...]
    w10 = w10_ref[...]
    f2 = (jnp.dot(h0, w10[:D_HID], preferred_element_type=jnp.float32)
          + jnp.dot(h1, w10[D_HID:], preferred_element_type=jnp.float32)
          + b10_ref[...])
    h0p_ref[...] = _ln(f2, s10_ref[...], o10_ref[...])
    w11 = w11_ref[...]
    y = (jnp.dot(h0, w11[:D_HID], preferred_element_type=jnp.float32)
         + jnp.dot(h1, w11[D_HID:], preferred_element_type=jnp.float32))
    for sidx in range(4):
        y_ref[sidx] = y[:, sidx * 128:(sidx + 1) * 128]


def _tcc_body(z_ref, h0p_ref, b11_ref, s11_ref, o11_ref, w2_ref, b2_ref,
              pred_ref):
    b11 = b11_ref[...]
    s11 = s11_ref[...]
    o11 = o11_ref[...]
    ts = [jnp.maximum(z_ref[s] + b11[:, s * 128:(s + 1) * 128], 0.0)
          for s in range(4)]
    sum1 = sum(jnp.sum(t, axis=1, keepdims=True) for t in ts)
    sum2 = sum(jnp.sum(t * t, axis=1, keepdims=True) for t in ts)
    mean = sum1 / D_HID
    var = sum2 / D_HID - mean * mean + 1e-9
    rstd = lax.rsqrt(var)
    h1ps = [(ts[s] - mean) * s11[:, s * 128:(s + 1) * 128] * rstd
            + o11[:, s * 128:(s + 1) * 128] for s in range(4)]
    h0p = h0p_ref[...]
    nsq = jnp.sum(h0p * h0p, axis=1, keepdims=True)
    for t in h1ps:
        nsq = nsq + jnp.sum(t * t, axis=1, keepdims=True)
    inv = 1.0 / jnp.maximum(jnp.sqrt(nsq), 1e-12)
    w2 = w2_ref[...]
    acc = jnp.dot(h0p, w2[:D_HID], preferred_element_type=jnp.float32)
    for s in range(4):
        acc = acc + jnp.dot(h1ps[s], w2[D_HID + s * 128:D_HID + (s + 1) * 128],
                            preferred_element_type=jnp.float32)
    pred_ref[...] = acc * inv + b2_ref[...]


def _row_spec(shape):
    nd = len(shape)
    return pl.BlockSpec(shape, lambda i: (i,) + (0,) * (nd - 1))


def _const_spec(shape):
    nd = len(shape)
    return pl.BlockSpec(shape, lambda i: (0,) * nd)


def _tca(x, lg, w, b, s, o, iota_row):
    return pl.pallas_call(
        _tca_body,
        grid=(GRID,),
        in_specs=[
            pl.BlockSpec((2, BLK, 128), lambda i: (0, i, 0)),
            _row_spec((BLK, _LG_W)),
            _const_spec((D_IN, D_HID)),
            _const_spec((1, D_HID)),
            _const_spec((1, D_HID)),
            _const_spec((1, D_HID)),
            _const_spec((1, _LG_W)),
        ],
        out_specs=[_row_spec((BLK, D_HID)), _row_spec((BLK, 1))],
        out_shape=[
            jax.ShapeDtypeStruct((N, D_HID), jnp.float32),
            jax.ShapeDtypeStruct((N, 1), jnp.float32),
        ],
    )(x, lg, w, b, s, o, iota_row)


def _tcb(x1s, h0, w01, b01, s01, o01, w10, b10, s10, o10, w11):
    return pl.pallas_call(
        _tcb_body,
        grid=(GRID,),
        in_specs=[
            pl.BlockSpec((2, BLK, 128), lambda i: (0, i, 0)),
            _row_spec((BLK, D_HID)),
            _const_spec((D_IN, D_HID)),
            _const_spec((1, D_HID)),
            _const_spec((1, D_HID)),
            _const_spec((1, D_HID)),
            _const_spec((2 * D_HID, D_HID)),
            _const_spec((1, D_HID)),
            _const_spec((1, D_HID)),
            _const_spec((1, D_HID)),
            _const_spec((2 * D_HID, D_HID)),
        ],
        out_specs=[_row_spec((BLK, D_HID)),
                   pl.BlockSpec((4, BLK, 128), lambda i: (0, i, 0))],
        out_shape=[
            jax.ShapeDtypeStruct((N, D_HID), jnp.float32),
            jax.ShapeDtypeStruct((4, N, 128), jnp.float32),
        ],
    )(x1s, h0, w01, b01, s01, o01, w10, b10, s10, o10, w11)


def _tcc(zs, h0p, b11, s11, o11, w2, b2):
    return pl.pallas_call(
        _tcc_body,
        grid=(GRID,),
        in_specs=[
            pl.BlockSpec((4, BLK, 128), lambda i: (0, i, 0)),
            _row_spec((BLK, D_HID)),
            _const_spec((1, D_HID)),
            _const_spec((1, D_HID)),
            _const_spec((1, D_HID)),
            _const_spec((2 * D_HID, 64)),
            _const_spec((1, 64)),
        ],
        out_specs=_row_spec((BLK, 64)),
        out_shape=jax.ShapeDtypeStruct((N, 64), jnp.float32),
    )(zs, h0p, b11, s11, o11, w2, b2)


def kernel(node_ids, edge_index, adj_vals, feats, labels,
           W0_0, b0_0, s0_0, o0_0, W0_1, b0_1, s0_1, o0_1,
           W1_0, b1_0, s1_0, o1_0, W1_1, b1_1, s1_1, o1_1,
           W2, b2):
    f32 = jnp.float32
    i32 = jnp.int32
    src = edge_index[1]
    dst = edge_index[0]
    npad = EP - E
    pad_idx = (jnp.arange(npad, dtype=i32) * 37) % N
    src1 = jnp.concatenate([src, pad_idx])
    dst3 = jnp.concatenate([dst, pad_idx]).reshape(EP // ECH, 1, ECH)
    val1 = jnp.concatenate([adj_vals, jnp.zeros((npad,), f32)])
    nid_pad = (jnp.arange(B_PAD - N, dtype=i32) * 41) % N
    nid1 = jnp.concatenate([node_ids, nid_pad])

    labels128 = jnp.pad(labels, ((0, 0), (0, 64)))
    x_pad, lg = _get_gather()(feats, labels128, nid1)

    iota_row = jnp.arange(_LG_W, dtype=f32).reshape(1, _LG_W)
    h0, am = _tca(x_pad, lg, W0_0, b0_0.reshape(1, -1), s0_0.reshape(1, -1),
                  o0_0.reshape(1, -1), iota_row)

    x1s = _make_spmm(2, B_PAD)(x_pad.reshape(2 * B_PAD, 128), src1, dst3,
                               val1)

    h0p, y = _tcb(x1s, h0, W0_1, b0_1.reshape(1, -1), s0_1.reshape(1, -1),
                  o0_1.reshape(1, -1), W1_0, b1_0.reshape(1, -1),
                  s1_0.reshape(1, -1), o1_0.reshape(1, -1), W1_1)

    zs = _make_spmm(4, N)(y.reshape(4 * N, 128), src1, dst3, val1)

    pred = _tcc(zs, h0p, b1_1.reshape(1, -1), s1_1.reshape(1, -1),
                o1_1.reshape(1, -1), W2, b2.reshape(1, -1))
    lab = am[:, 0].astype(i32)
    return (pred, lab)
